# baseline (speedup 1.0000x reference)
import functools

import jax
import jax.numpy as jnp
from jax import lax
from jax.experimental import pallas as pl
from jax.experimental.pallas import tpu as pltpu
from jax.experimental.pallas import tpu_sc as plsc

E = 16
D = 768
I = 4 * 768
SH = 2 * 768
S = 2048
BT = 256
BTSH = 8
SPAD = S + E * BT
G = SPAD // BT
NC = 2
NS = 16
NT = NC * NS
TPW = S // NT
TPC = TPW // 2
WL = 128


@functools.cache
def _mesh():
    return plsc.VectorSubcoreMesh(core_axis_name="c", subcore_axis_name="s")


def _router_body(x_ref, gw_ref, w_ref, pos_ref, be_ref, act_ref):
    x = x_ref[...]
    logits = lax.dot_general(x, gw_ref[...], (((1,), (1,)), ((), ())),
                             preferred_element_type=jnp.float32)
    m = jnp.max(logits, axis=1, keepdims=True)
    w1 = 1.0 / jnp.sum(jnp.exp(logits - m), axis=1, keepdims=True)
    w_ref[...] = jnp.broadcast_to(w1, (S, WL))
    iota_e = lax.broadcasted_iota(jnp.int32, (S, E), 1)
    ids2 = jnp.min(jnp.where(logits == m, iota_e, jnp.int32(1 << 30)),
                   axis=1, keepdims=True)
    onehot = (iota_e == ids2).astype(jnp.int32)
    counts = jnp.sum(onehot, axis=0, keepdims=True)
    r = onehot
    sh = 1
    while sh < S:
        r = r + jnp.concatenate(
            [jnp.zeros((sh, E), jnp.int32), r[:S - sh]], axis=0)
        sh *= 2
    rank01 = r - onehot
    pad_tot = ((counts + (BT - 1)) >> BTSH) << BTSH
    c = pad_tot
    sh = 1
    while sh < E:
        c = c + jnp.concatenate(
            [jnp.zeros((1, sh), jnp.int32), c[:, :E - sh]], axis=1)
        sh *= 2
    csum = c
    pad_excl = csum - pad_tot
    pos_ref[...] = jnp.sum(onehot * (pad_excl + rank01), axis=1, keepdims=True)
    iota_g = lax.broadcasted_iota(jnp.int32, (G, E), 0)
    cnt = jnp.sum((iota_g * BT >= csum).astype(jnp.int32), axis=1, keepdims=True)
    iota_e2 = lax.broadcasted_iota(jnp.int32, (1, E), 1)
    mlast = jnp.max(jnp.where(counts > 0, iota_e2, 0), axis=1, keepdims=True)
    be_ref[...] = jnp.minimum(cnt, mlast)
    total_padded = csum[:, E - 1:E]
    iota_g1 = lax.broadcasted_iota(jnp.int32, (G, 1), 0)
    act_ref[...] = (iota_g1 * BT < total_padded).astype(jnp.int32)


def _router(x, gate_w):
    return pl.pallas_call(
        _router_body,
        out_shape=(jax.ShapeDtypeStruct((S, WL), jnp.float32),
                   jax.ShapeDtypeStruct((S, 1), jnp.int32),
                   jax.ShapeDtypeStruct((G, 1), jnp.int32),
                   jax.ShapeDtypeStruct((G, 1), jnp.int32)),
    )(x, gate_w)


@functools.cache
def _make_dispatch():
    return functools.partial(
        pl.kernel,
        out_type=(jax.ShapeDtypeStruct((SPAD, D), jnp.float32),
                  jax.ShapeDtypeStruct((SPAD, WL), jnp.float32)),
        mesh=_mesh(),
        scratch_types=[
            pltpu.VMEM((TPC,), jnp.int32),
            pltpu.VMEM((TPC,), jnp.int32),
            pltpu.VMEM((TPC, D), jnp.float32),
            pltpu.VMEM((TPC, D), jnp.float32),
            pltpu.VMEM((TPC, WL), jnp.float32),
            pltpu.VMEM((TPC, WL), jnp.float32),
            pltpu.SemaphoreType.DMA,
            pltpu.SemaphoreType.DMA,
            pltpu.SemaphoreType.DMA,
            pltpu.SemaphoreType.DMA,
            pltpu.SemaphoreType.DMA,
            pltpu.SemaphoreType.DMA,
        ],
    )(_dispatch_body)


def _dispatch_body(pos_hbm, x_hbm, w_hbm, xs_hbm, ws_hbm,
                   pos0, pos1, rows0, rows1, wrows0, wrows1,
                   smp0, smp1, smx0, smx1, smw0, smw1):
    wid = lax.axis_index("s") * NC + lax.axis_index("c")
    base = wid * TPW
    ap0 = pltpu.async_copy(pos_hbm.at[pl.ds(base, TPC)], pos0, smp0)
    ap1 = pltpu.async_copy(pos_hbm.at[pl.ds(base + TPC, TPC)], pos1, smp1)
    ax0 = pltpu.async_copy(x_hbm.at[pl.ds(base, TPC)], rows0, smx0)
    ax1 = pltpu.async_copy(x_hbm.at[pl.ds(base + TPC, TPC)], rows1, smx1)
    aw0 = pltpu.async_copy(w_hbm.at[pl.ds(base, TPC)], wrows0, smw0)
    aw1 = pltpu.async_copy(w_hbm.at[pl.ds(base + TPC, TPC)], wrows1, smw1)
    ap0.wait()
    ax0.wait()
    s0 = pltpu.async_copy(rows0, xs_hbm.at[pos0], smx0)
    ap1.wait()
    ax1.wait()
    s1 = pltpu.async_copy(rows1, xs_hbm.at[pos1], smx1)
    aw0.wait()
    sw0 = pltpu.async_copy(wrows0, ws_hbm.at[pos0], smw0)
    aw1.wait()
    sw1 = pltpu.async_copy(wrows1, ws_hbm.at[pos1], smw1)
    s0.wait()
    s1.wait()
    sw0.wait()
    sw1.wait()


IBLK = I // 2
K = 2


def _mlp_body(be_ref, act_ref, x_ref, w_ref, wg_ref, wu_ref, wd_ref,
              sg_any, su_any, sd_any, o_ref,
              acc_ref, sgs, sus, sds, sgb, sub, sdb, sem1, sem2, sem3):
    k = pl.program_id(0)
    gidx = pl.program_id(1)

    @pl.when((k == 0) & (gidx == 0))
    def _():
        pltpu.make_async_copy(sg_any, sgs, sem1).start()
        pltpu.make_async_copy(su_any, sus, sem2).start()
        pltpu.make_async_copy(sd_any, sds, sem3).start()

    @pl.when((k == 1) & (gidx == 0))
    def _():
        pltpu.make_async_copy(sg_any, sgs, sem1).wait()
        pltpu.make_async_copy(su_any, sus, sem2).wait()
        pltpu.make_async_copy(sd_any, sds, sem3).wait()
        sgb[...] = sgs[...].astype(jnp.bfloat16)
        sub[...] = sus[...].astype(jnp.bfloat16)
        sdb[...] = sds[...].astype(jnp.bfloat16)

    @pl.when(act_ref[gidx] > 0)
    def _():
        x = x_ref[...]
        g = lax.dot_general(x, wg_ref[0], (((1,), (1,)), ((), ())),
                            preferred_element_type=jnp.float32)
        u = lax.dot_general(x, wu_ref[0], (((1,), (1,)), ((), ())),
                            preferred_element_type=jnp.float32)
        h = g * jax.nn.sigmoid(g) * u
        part = lax.dot_general(h, wd_ref[0], (((1,), (1,)), ((), ())),
                               preferred_element_type=jnp.float32)
        wcol = w_ref[:, :1]
        rows = pl.ds(pl.multiple_of(gidx * BT, BT), BT)

        @pl.when(k == 0)
        def _():
            acc_ref[rows, :] = (wcol * part).astype(jnp.bfloat16)

        @pl.when(k == 1)
        def _():
            xb = x.astype(jnp.bfloat16)
            sg = lax.dot_general(xb, sgb[...], (((1,), (1,)), ((), ())),
                                 preferred_element_type=jnp.float32)
            su = lax.dot_general(xb, sub[...], (((1,), (1,)), ((), ())),
                                 preferred_element_type=jnp.float32)
            hs = (sg * jax.nn.sigmoid(sg) * su).astype(jnp.bfloat16)
            ysh = lax.dot_general(hs, sdb[...], (((1,), (1,)), ((), ())),
                                  preferred_element_type=jnp.float32)
            o_ref[...] = (acc_ref[rows, :].astype(jnp.float32)
                          + wcol * part + ysh)


def _grouped_mlp(block_expert, block_act, xs, ws, Wg, Wu, Wd, Sg, Su, Sd):
    grid_spec = pltpu.PrefetchScalarGridSpec(
        num_scalar_prefetch=2,
        grid=(K, G),
        in_specs=[
            pl.BlockSpec((BT, D),
                         lambda k, g, be, act: (jnp.where(act[g] > 0, g, 0), 0)),
            pl.BlockSpec((BT, WL),
                         lambda k, g, be, act: (jnp.where(act[g] > 0, g, 0), 0)),
            pl.BlockSpec((1, IBLK, D), lambda k, g, be, act: (be[g], k, 0)),
            pl.BlockSpec((1, IBLK, D), lambda k, g, be, act: (be[g], k, 0)),
            pl.BlockSpec((1, D, IBLK), lambda k, g, be, act: (be[g], 0, k)),
            pl.BlockSpec(memory_space=pl.ANY),
            pl.BlockSpec(memory_space=pl.ANY),
            pl.BlockSpec(memory_space=pl.ANY),
        ],
        out_specs=pl.BlockSpec(
            (BT, D),
            lambda k, g, be, act: (
                jnp.where((k == 1) & (act[g] > 0), g, G), 0)),
        scratch_shapes=[
            pltpu.VMEM((SPAD, D), jnp.bfloat16),
            pltpu.VMEM((SH, D), jnp.float32),
            pltpu.VMEM((SH, D), jnp.float32),
            pltpu.VMEM((D, SH), jnp.float32),
            pltpu.VMEM((SH, D), jnp.bfloat16),
            pltpu.VMEM((SH, D), jnp.bfloat16),
            pltpu.VMEM((D, SH), jnp.bfloat16),
            pltpu.SemaphoreType.DMA,
            pltpu.SemaphoreType.DMA,
            pltpu.SemaphoreType.DMA,
        ],
    )
    return pl.pallas_call(
        _mlp_body,
        grid_spec=grid_spec,
        out_shape=jax.ShapeDtypeStruct((SPAD + BT, D), jnp.float32),
        compiler_params=pltpu.CompilerParams(
            dimension_semantics=("arbitrary", "arbitrary"),
            vmem_limit_bytes=128 * 1024 * 1024,
        ),
    )(block_expert, block_act, xs, ws, Wg, Wu, Wd, Sg, Su, Sd)


@functools.cache
def _make_gather_back():
    return functools.partial(
        pl.kernel,
        out_type=jax.ShapeDtypeStruct((S, D), jnp.float32),
        mesh=_mesh(),
        scratch_types=[
            pltpu.VMEM((TPC,), jnp.int32),
            pltpu.VMEM((TPC,), jnp.int32),
            pltpu.VMEM((TPC, D), jnp.float32),
            pltpu.VMEM((TPC, D), jnp.float32),
            pltpu.SemaphoreType.DMA,
            pltpu.SemaphoreType.DMA,
            pltpu.SemaphoreType.DMA,
            pltpu.SemaphoreType.DMA,
        ],
    )(_gather_back_body)


def _gather_back_body(pos_hbm, os_hbm, out_hbm,
                      pos0, pos1, rows0, rows1, smp0, smp1, smr0, smr1):
    wid = lax.axis_index("s") * NC + lax.axis_index("c")
    base = wid * TPW
    ap0 = pltpu.async_copy(pos_hbm.at[pl.ds(base, TPC)], pos0, smp0)
    ap1 = pltpu.async_copy(pos_hbm.at[pl.ds(base + TPC, TPC)], pos1, smp1)
    ap0.wait()
    g0 = pltpu.async_copy(os_hbm.at[pos0], rows0, smr0)
    ap1.wait()
    g1 = pltpu.async_copy(os_hbm.at[pos1], rows1, smr1)
    g0.wait()
    o0 = pltpu.async_copy(rows0, out_hbm.at[pl.ds(base, TPC)], smr0)
    g1.wait()
    o1 = pltpu.async_copy(rows1, out_hbm.at[pl.ds(base + TPC, TPC)], smr1)
    o0.wait()
    o1.wait()


def kernel(hidden_states, t, gate_w, Wg, Wu, Wd, Sg, Su, Sd):
    x = hidden_states.reshape(S, D)
    w2d, pos2d, be2d, act2d = _router(x, gate_w)
    pos = pos2d.reshape(S)
    xs, ws = _make_dispatch()(pos, x, w2d)
    os_ = _grouped_mlp(be2d.reshape(G), act2d.reshape(G), xs, ws,
                       Wg, Wu, Wd, Sg, Su, Sd)
    y = _make_gather_back()(pos, os_)
    return y.reshape(1, S, D)


# speedup vs baseline: 1.0020x; 1.0020x over previous
import functools

import jax
import jax.numpy as jnp
from jax import lax
from jax.experimental import pallas as pl
from jax.experimental.pallas import tpu as pltpu
from jax.experimental.pallas import tpu_sc as plsc

E = 16
D = 768
I = 4 * 768
SH = 2 * 768
S = 2048
BT = 256
BTSH = 8
SPAD = S + E * BT
G = SPAD // BT
NC = 2
NS = 16
NT = NC * NS
TPW = S // NT
TPC = TPW // 2
WL = 128


@functools.cache
def _mesh():
    return plsc.VectorSubcoreMesh(core_axis_name="c", subcore_axis_name="s")


def _router_body(x_ref, gw_ref, w_ref, pos_ref, be_ref, act_ref):
    x = x_ref[...]
    logits = lax.dot_general(x, gw_ref[...], (((1,), (1,)), ((), ())),
                             preferred_element_type=jnp.float32)
    m = jnp.max(logits, axis=1, keepdims=True)
    w1 = 1.0 / jnp.sum(jnp.exp(logits - m), axis=1, keepdims=True)
    w_ref[...] = jnp.broadcast_to(w1, (S, WL))
    iota_e = lax.broadcasted_iota(jnp.int32, (S, E), 1)
    ids2 = jnp.min(jnp.where(logits == m, iota_e, jnp.int32(1 << 30)),
                   axis=1, keepdims=True)
    onehot = (iota_e == ids2).astype(jnp.int32)
    counts = jnp.sum(onehot, axis=0, keepdims=True)
    r = onehot
    sh = 1
    while sh < S:
        r = r + jnp.concatenate(
            [jnp.zeros((sh, E), jnp.int32), r[:S - sh]], axis=0)
        sh *= 2
    rank01 = r - onehot
    pad_tot = ((counts + (BT - 1)) >> BTSH) << BTSH
    c = pad_tot
    sh = 1
    while sh < E:
        c = c + jnp.concatenate(
            [jnp.zeros((1, sh), jnp.int32), c[:, :E - sh]], axis=1)
        sh *= 2
    csum = c
    pad_excl = csum - pad_tot
    pos_ref[...] = jnp.sum(onehot * (pad_excl + rank01), axis=1, keepdims=True)
    iota_g = lax.broadcasted_iota(jnp.int32, (G, E), 0)
    cnt = jnp.sum((iota_g * BT >= csum).astype(jnp.int32), axis=1, keepdims=True)
    iota_e2 = lax.broadcasted_iota(jnp.int32, (1, E), 1)
    mlast = jnp.max(jnp.where(counts > 0, iota_e2, 0), axis=1, keepdims=True)
    be_ref[...] = jnp.minimum(cnt, mlast)
    total_padded = csum[:, E - 1:E]
    iota_g1 = lax.broadcasted_iota(jnp.int32, (G, 1), 0)
    act_ref[...] = (iota_g1 * BT < total_padded).astype(jnp.int32)


def _router(x, gate_w):
    return pl.pallas_call(
        _router_body,
        out_shape=(jax.ShapeDtypeStruct((S, WL), jnp.float32),
                   jax.ShapeDtypeStruct((S, 1), jnp.int32),
                   jax.ShapeDtypeStruct((G, 1), jnp.int32),
                   jax.ShapeDtypeStruct((G, 1), jnp.int32)),
    )(x, gate_w)


@functools.cache
def _make_dispatch():
    return functools.partial(
        pl.kernel,
        out_type=(jax.ShapeDtypeStruct((SPAD, D), jnp.float32),
                  jax.ShapeDtypeStruct((SPAD, WL), jnp.float32)),
        mesh=_mesh(),
        scratch_types=[
            pltpu.VMEM((TPC,), jnp.int32),
            pltpu.VMEM((TPC,), jnp.int32),
            pltpu.VMEM((TPC, D), jnp.float32),
            pltpu.VMEM((TPC, D), jnp.float32),
            pltpu.VMEM((TPC, WL), jnp.float32),
            pltpu.VMEM((TPC, WL), jnp.float32),
            pltpu.SemaphoreType.DMA,
            pltpu.SemaphoreType.DMA,
            pltpu.SemaphoreType.DMA,
            pltpu.SemaphoreType.DMA,
            pltpu.SemaphoreType.DMA,
            pltpu.SemaphoreType.DMA,
        ],
    )(_dispatch_body)


def _dispatch_body(pos_hbm, x_hbm, w_hbm, xs_hbm, ws_hbm,
                   pos0, pos1, rows0, rows1, wrows0, wrows1,
                   smp0, smp1, smx0, smx1, smw0, smw1):
    wid = lax.axis_index("s") * NC + lax.axis_index("c")
    base = wid * TPW
    ap0 = pltpu.async_copy(pos_hbm.at[pl.ds(base, TPC)], pos0, smp0)
    ap1 = pltpu.async_copy(pos_hbm.at[pl.ds(base + TPC, TPC)], pos1, smp1)
    ax0 = pltpu.async_copy(x_hbm.at[pl.ds(base, TPC)], rows0, smx0)
    ax1 = pltpu.async_copy(x_hbm.at[pl.ds(base + TPC, TPC)], rows1, smx1)
    aw0 = pltpu.async_copy(w_hbm.at[pl.ds(base, TPC)], wrows0, smw0)
    aw1 = pltpu.async_copy(w_hbm.at[pl.ds(base + TPC, TPC)], wrows1, smw1)
    ap0.wait()
    ax0.wait()
    s0 = pltpu.async_copy(rows0, xs_hbm.at[pos0], smx0)
    ap1.wait()
    ax1.wait()
    s1 = pltpu.async_copy(rows1, xs_hbm.at[pos1], smx1)
    aw0.wait()
    sw0 = pltpu.async_copy(wrows0, ws_hbm.at[pos0], smw0)
    aw1.wait()
    sw1 = pltpu.async_copy(wrows1, ws_hbm.at[pos1], smw1)
    s0.wait()
    s1.wait()
    sw0.wait()
    sw1.wait()


IBLK = I // 2
K = 2


def _mlp_body(be_ref, act_ref, x_ref, w_ref, wg_ref, wu_ref, wd_ref,
              sg_any, su_any, sd_any, o_ref,
              acc_ref, sgs, sus, sds, sem1, sem2, sem3):
    k = pl.program_id(0)
    gidx = pl.program_id(1)

    @pl.when((k == 0) & (gidx == 0))
    def _():
        pltpu.make_async_copy(sg_any, sgs, sem1).start()
        pltpu.make_async_copy(su_any, sus, sem2).start()
        pltpu.make_async_copy(sd_any, sds, sem3).start()

    @pl.when((k == 1) & (gidx == 0))
    def _():
        pltpu.make_async_copy(sg_any, sgs, sem1).wait()
        pltpu.make_async_copy(su_any, sus, sem2).wait()
        pltpu.make_async_copy(sd_any, sds, sem3).wait()

    @pl.when(act_ref[gidx] > 0)
    def _():
        x = x_ref[...]
        g = lax.dot_general(x, wg_ref[0], (((1,), (1,)), ((), ())),
                            preferred_element_type=jnp.float32)
        u = lax.dot_general(x, wu_ref[0], (((1,), (1,)), ((), ())),
                            preferred_element_type=jnp.float32)
        h = g * jax.nn.sigmoid(g) * u
        part = lax.dot_general(h, wd_ref[0], (((1,), (1,)), ((), ())),
                               preferred_element_type=jnp.float32)
        wcol = w_ref[:, :1]
        rows = pl.ds(pl.multiple_of(gidx * BT, BT), BT)

        @pl.when(k == 0)
        def _():
            acc_ref[rows, :] = (wcol * part).astype(jnp.bfloat16)

        @pl.when(k == 1)
        def _():
            sg = lax.dot_general(x, sgs[...], (((1,), (1,)), ((), ())),
                                 preferred_element_type=jnp.float32)
            su = lax.dot_general(x, sus[...], (((1,), (1,)), ((), ())),
                                 preferred_element_type=jnp.float32)
            hs = sg * jax.nn.sigmoid(sg) * su
            ysh = lax.dot_general(hs, sds[...], (((1,), (1,)), ((), ())),
                                  preferred_element_type=jnp.float32)
            o_ref[...] = (acc_ref[rows, :].astype(jnp.float32)
                          + wcol * part + ysh)


def _grouped_mlp(block_expert, block_act, xs, ws, Wg, Wu, Wd, Sg, Su, Sd):
    grid_spec = pltpu.PrefetchScalarGridSpec(
        num_scalar_prefetch=2,
        grid=(K, G),
        in_specs=[
            pl.BlockSpec((BT, D),
                         lambda k, g, be, act: (jnp.where(act[g] > 0, g, 0), 0)),
            pl.BlockSpec((BT, WL),
                         lambda k, g, be, act: (jnp.where(act[g] > 0, g, 0), 0)),
            pl.BlockSpec((1, IBLK, D), lambda k, g, be, act: (be[g], k, 0)),
            pl.BlockSpec((1, IBLK, D), lambda k, g, be, act: (be[g], k, 0)),
            pl.BlockSpec((1, D, IBLK), lambda k, g, be, act: (be[g], 0, k)),
            pl.BlockSpec(memory_space=pl.ANY),
            pl.BlockSpec(memory_space=pl.ANY),
            pl.BlockSpec(memory_space=pl.ANY),
        ],
        out_specs=pl.BlockSpec(
            (BT, D),
            lambda k, g, be, act: (
                jnp.where((k == 1) & (act[g] > 0), g, G), 0)),
        scratch_shapes=[
            pltpu.VMEM((SPAD, D), jnp.bfloat16),
            pltpu.VMEM((SH, D), jnp.float32),
            pltpu.VMEM((SH, D), jnp.float32),
            pltpu.VMEM((D, SH), jnp.float32),
            pltpu.SemaphoreType.DMA,
            pltpu.SemaphoreType.DMA,
            pltpu.SemaphoreType.DMA,
        ],
    )
    return pl.pallas_call(
        _mlp_body,
        grid_spec=grid_spec,
        out_shape=jax.ShapeDtypeStruct((SPAD + BT, D), jnp.float32),
        compiler_params=pltpu.CompilerParams(
            dimension_semantics=("arbitrary", "arbitrary"),
            vmem_limit_bytes=128 * 1024 * 1024,
        ),
    )(block_expert, block_act, xs, ws, Wg, Wu, Wd, Sg, Su, Sd)


@functools.cache
def _make_gather_back():
    return functools.partial(
        pl.kernel,
        out_type=jax.ShapeDtypeStruct((S, D), jnp.float32),
        mesh=_mesh(),
        scratch_types=[
            pltpu.VMEM((TPC,), jnp.int32),
            pltpu.VMEM((TPC,), jnp.int32),
            pltpu.VMEM((TPC, D), jnp.float32),
            pltpu.VMEM((TPC, D), jnp.float32),
            pltpu.SemaphoreType.DMA,
            pltpu.SemaphoreType.DMA,
            pltpu.SemaphoreType.DMA,
            pltpu.SemaphoreType.DMA,
        ],
    )(_gather_back_body)


def _gather_back_body(pos_hbm, os_hbm, out_hbm,
                      pos0, pos1, rows0, rows1, smp0, smp1, smr0, smr1):
    wid = lax.axis_index("s") * NC + lax.axis_index("c")
    base = wid * TPW
    ap0 = pltpu.async_copy(pos_hbm.at[pl.ds(base, TPC)], pos0, smp0)
    ap1 = pltpu.async_copy(pos_hbm.at[pl.ds(base + TPC, TPC)], pos1, smp1)
    ap0.wait()
    g0 = pltpu.async_copy(os_hbm.at[pos0], rows0, smr0)
    ap1.wait()
    g1 = pltpu.async_copy(os_hbm.at[pos1], rows1, smr1)
    g0.wait()
    o0 = pltpu.async_copy(rows0, out_hbm.at[pl.ds(base, TPC)], smr0)
    g1.wait()
    o1 = pltpu.async_copy(rows1, out_hbm.at[pl.ds(base + TPC, TPC)], smr1)
    o0.wait()
    o1.wait()


def kernel(hidden_states, t, gate_w, Wg, Wu, Wd, Sg, Su, Sd):
    x = hidden_states.reshape(S, D)
    w2d, pos2d, be2d, act2d = _router(x, gate_w)
    pos = pos2d.reshape(S)
    xs, ws = _make_dispatch()(pos, x, w2d)
    os_ = _grouped_mlp(be2d.reshape(G), act2d.reshape(G), xs, ws,
                       Wg, Wu, Wd, Sg, Su, Sd)
    y = _make_gather_back()(pos, os_)
    return y.reshape(1, S, D)
